# Initial kernel scaffold; baseline (speedup 1.0000x reference)
import jax
import jax.numpy as jnp
from jax.experimental import pallas as pl


def kernel(confidence, predicted_locations, gts, counts, anchors):
    raise NotImplementedError("write your pallas kernel here")


# speedup vs baseline: 5.4056x; 5.4056x over previous
import functools

import jax
import jax.numpy as jnp
from jax.experimental import pallas as pl
from jax.experimental.pallas import tpu as pltpu

POS_TH = 0.5
NEG_TH = 0.4
EPS = 1e-7
BETA = 1.0 / 9.0

_INTERPRET = False

T1 = 2048
T2 = 1024
NGT = 50


def _match_body(a_ref, gt_ref, conf_ref, mbox_ref):
    ax1 = a_ref[0:1, :]
    ay1 = a_ref[1:2, :]
    ax2 = a_ref[2:3, :]
    ay2 = a_ref[3:4, :]
    area_a = (ax2 - ax1) * (ay2 - ay1)

    def step(g, carry):
        best, lab, mx1, my1, mx2, my2 = carry
        gx1 = gt_ref[0, 0, g]
        gy1 = gt_ref[0, 1, g]
        gx2 = gt_ref[0, 2, g]
        gy2 = gt_ref[0, 3, g]
        garea = gt_ref[0, 4, g]
        glab = gt_ref[0, 5, g]
        wx = jnp.maximum(jnp.minimum(ax2, gx2) - jnp.maximum(ax1, gx1), 0.0)
        wy = jnp.maximum(jnp.minimum(ay2, gy2) - jnp.maximum(ay1, gy1), 0.0)
        inter = wx * wy
        iou = inter / (area_a + (garea - inter))
        upd = iou > best
        best = jnp.where(upd, iou, best)
        lab = jnp.where(upd, glab, lab)
        mx1 = jnp.where(upd, gx1, mx1)
        my1 = jnp.where(upd, gy1, my1)
        mx2 = jnp.where(upd, gx2, mx2)
        my2 = jnp.where(upd, gy2, my2)
        return best, lab, mx1, my1, mx2, my2

    z = jnp.zeros((1, T1), jnp.float32)
    init = (jnp.full((1, T1), -1.0, jnp.float32), z, z, z, z, z)
    best, lab, mx1, my1, mx2, my2 = jax.lax.fori_loop(0, NGT, step, init)
    conf = jnp.where(best < POS_TH, -1.0, lab)
    conf = jnp.where(best < NEG_TH, 0.0, conf)
    conf_ref[0, 0:1, :] = conf
    mbox_ref[0, 0:1, :] = mx1
    mbox_ref[0, 1:2, :] = my1
    mbox_ref[0, 2:3, :] = mx2
    mbox_ref[0, 3:4, :] = my2


def _loss_body(nsteps, conf_cls_ref, ploc_ref, mbox_ref, lab_ref, anch_ref,
               out1_ref, out2_ref, acc_ref):
    b = pl.program_id(0)
    t = pl.program_id(1)
    first = jnp.logical_and(b == 0, t == 0)
    last = jnp.logical_and(b == pl.num_programs(0) - 1,
                           t == pl.num_programs(1) - 1)

    @pl.when(first)
    def _():
        acc_ref[0] = 0.0
        acc_ref[1] = 0.0
        acc_ref[2] = 0.0
        acc_ref[3] = 0.0

    lab = lab_ref[0]
    pos = (lab > 0.0).astype(jnp.float32)
    neg = (lab == 0.0).astype(jnp.float32)
    np_part = jnp.sum(pos)

    m = mbox_ref[0]
    a = anch_ref[...]
    mcx = (m[:, 0:2] + m[:, 2:4]) * 0.5
    mwh = jnp.maximum(m[:, 2:4] - m[:, 0:2], 1e-6)
    loc12 = (mcx - a[:, 0:2]) / (a[:, 2:4] * 0.1)
    loc34 = jnp.log(mwh / a[:, 2:4]) / 0.2
    loc = jnp.concatenate([loc12, loc34], axis=1)
    n = jnp.abs(ploc_ref[0] - loc)
    sl1 = jnp.where(n < BETA, 0.5 * n * n / BETA, n - 0.5 * BETA)
    sl1_part = jnp.sum(sl1 * pos)

    x0 = conf_cls_ref[0, :, 0:1]
    p0 = jnp.clip(1.0 / (1.0 + jnp.exp(-x0)), EPS, 1.0 - EPS)
    bin_part = jnp.sum(pos * (-jnp.log(p0)) + 0.5 * neg * (-jnp.log(1.0 - p0)))

    x = conf_cls_ref[0, :, 1:81]
    p = jnp.clip(1.0 / (1.0 + jnp.exp(-x)), EPS, 1.0 - EPS)
    cls = (jax.lax.broadcasted_iota(jnp.int32, (T2, 80), 1) + 1
           ).astype(jnp.float32)
    q = jnp.where(lab == cls, p, 1.0 - p)
    cls_part = jnp.sum(-jnp.log(q) * pos)

    acc_ref[0] += np_part
    acc_ref[1] += sl1_part
    acc_ref[2] += bin_part
    acc_ref[3] += cls_part

    @pl.when(last)
    def _():
        num_pos = jnp.maximum(1.0, acc_ref[0])
        out1_ref[...] = jnp.full((1, 1), acc_ref[1] / (num_pos * 4.0))
        out2_ref[...] = jnp.full((1, 1), (acc_ref[2] + acc_ref[3])
                                 / (2.0 * num_pos))


def kernel(confidence, predicted_locations, gts, counts, anchors):
    B, A, NC1 = confidence.shape

    corners = jnp.concatenate(
        [anchors[:, :2] - anchors[:, 2:] * 0.5,
         anchors[:, :2] + anchors[:, 2:] * 0.5], axis=1)
    anchors8 = jnp.concatenate(
        [corners.T, jnp.zeros((4, A), jnp.float32)], axis=0)

    valid = (jnp.arange(NGT)[None, :] < counts[:, None])
    sent = jnp.array([-5.0, -5.0, -4.0, -4.0], jnp.float32)
    gbox = jnp.where(valid[:, :, None], gts[:, :, :4], sent[None, None, :])
    garea = ((gbox[:, :, 2] - gbox[:, :, 0])
             * (gbox[:, :, 3] - gbox[:, :, 1]) + 1e-9)
    glab = jnp.where(valid, gts[:, :, 4], 0.0)
    gt_t = jnp.concatenate(
        [jnp.swapaxes(gbox, 1, 2), garea[:, None, :], glab[:, None, :],
         jnp.zeros((B, 2, NGT), jnp.float32)], axis=1)

    conf_l, mbox_l = pl.pallas_call(
        _match_body,
        grid=(B, A // T1),
        in_specs=[
            pl.BlockSpec((8, T1), lambda b, t: (0, t)),
            pl.BlockSpec((1, 8, NGT), lambda b, t: (b, 0, 0),
                         memory_space=pltpu.SMEM),
        ],
        out_specs=[
            pl.BlockSpec((1, 1, T1), lambda b, t: (b, 0, t)),
            pl.BlockSpec((1, 4, T1), lambda b, t: (b, 0, t)),
        ],
        out_shape=[
            jax.ShapeDtypeStruct((B, 1, A), jnp.float32),
            jax.ShapeDtypeStruct((B, 4, A), jnp.float32),
        ],
        interpret=_INTERPRET,
    )(anchors8, gt_t)

    mbox_s = jnp.swapaxes(mbox_l, 1, 2)
    lab_s = jnp.swapaxes(conf_l, 1, 2)

    nsteps = (B, A // T2)
    out1, out2 = pl.pallas_call(
        functools.partial(_loss_body, nsteps),
        grid=nsteps,
        in_specs=[
            pl.BlockSpec((1, T2, NC1), lambda b, t: (b, t, 0)),
            pl.BlockSpec((1, T2, 4), lambda b, t: (b, t, 0)),
            pl.BlockSpec((1, T2, 4), lambda b, t: (b, t, 0)),
            pl.BlockSpec((1, T2, 1), lambda b, t: (b, t, 0)),
            pl.BlockSpec((T2, 4), lambda b, t: (t, 0)),
        ],
        out_specs=[
            pl.BlockSpec((1, 1), lambda b, t: (0, 0)),
            pl.BlockSpec((1, 1), lambda b, t: (0, 0)),
        ],
        out_shape=[
            jax.ShapeDtypeStruct((1, 1), jnp.float32),
            jax.ShapeDtypeStruct((1, 1), jnp.float32),
        ],
        scratch_shapes=[pltpu.SMEM((4,), jnp.float32)],
        interpret=_INTERPRET,
    )(confidence, predicted_locations, mbox_s, lab_s, anchors)

    return out1[0, 0], out2[0, 0]


# speedup vs baseline: 12.1557x; 2.2487x over previous
import jax
import jax.numpy as jnp
from jax.experimental import pallas as pl
from jax.experimental.pallas import tpu as pltpu

POS_TH = 0.5
NEG_TH = 0.4
EPS = 1e-7
BETA = 1.0 / 9.0

_INTERPRET = False

T1 = 4096
T2 = 2048
NGT = 50


def _match_body(a_ref, gt_ref, ploc_ref, conf_ref, scal_ref, acc_ref):
    b = pl.program_id(0)
    t = pl.program_id(1)
    first = jnp.logical_and(b == 0, t == 0)
    last = jnp.logical_and(b == pl.num_programs(0) - 1,
                           t == pl.num_programs(1) - 1)

    @pl.when(first)
    def _():
        acc_ref[0] = 0.0
        acc_ref[1] = 0.0

    ax1 = a_ref[0:1, :]
    ay1 = a_ref[1:2, :]
    ax2 = a_ref[2:3, :]
    ay2 = a_ref[3:4, :]
    area_a = (ax2 - ax1) * (ay2 - ay1)

    z = jnp.zeros((1, T1), jnp.float32)
    best = jnp.full((1, T1), -1.0, jnp.float32)
    lab = z
    mx1 = z
    my1 = z
    mx2 = z
    my2 = z
    for g in range(NGT):
        gx1 = gt_ref[0, 0, g]
        gy1 = gt_ref[0, 1, g]
        gx2 = gt_ref[0, 2, g]
        gy2 = gt_ref[0, 3, g]
        garea = gt_ref[0, 4, g]
        glab = gt_ref[0, 5, g]
        wx = jnp.maximum(jnp.minimum(ax2, gx2) - jnp.maximum(ax1, gx1), 0.0)
        wy = jnp.maximum(jnp.minimum(ay2, gy2) - jnp.maximum(ay1, gy1), 0.0)
        inter = wx * wy
        iou = inter / (area_a + (garea - inter))
        upd = iou > best
        best = jnp.where(upd, iou, best)
        lab = jnp.where(upd, glab, lab)
        mx1 = jnp.where(upd, gx1, mx1)
        my1 = jnp.where(upd, gy1, my1)
        mx2 = jnp.where(upd, gx2, mx2)
        my2 = jnp.where(upd, gy2, my2)

    conf = jnp.where(best < POS_TH, -1.0, lab)
    conf = jnp.where(best < NEG_TH, 0.0, conf)
    conf_ref[0, 0:1, :] = conf
    pos = (conf > 0.0).astype(jnp.float32)

    acx = a_ref[4:5, :]
    acy = a_ref[5:6, :]
    aw = a_ref[6:7, :]
    ah = a_ref[7:8, :]
    l0 = ((mx1 + mx2) * 0.5 - acx) / (aw * 0.1)
    l1 = ((my1 + my2) * 0.5 - acy) / (ah * 0.1)
    l2 = jnp.log(jnp.maximum(mx2 - mx1, 1e-6) / aw) * 5.0
    l3 = jnp.log(jnp.maximum(my2 - my1, 1e-6) / ah) * 5.0

    sl1 = z
    for i, l in enumerate((l0, l1, l2, l3)):
        n = jnp.abs(ploc_ref[0, i:i + 1, :] - l)
        sl1 = sl1 + jnp.where(n < BETA, n * n * (0.5 / BETA), n - 0.5 * BETA)
    acc_ref[0] += jnp.sum(sl1 * pos)
    acc_ref[1] += jnp.sum(pos)

    @pl.when(last)
    def _():
        scal_ref[0:1, 0:1] = jnp.full((1, 1), acc_ref[0])
        scal_ref[0:1, 1:2] = jnp.full((1, 1), acc_ref[1])


def _loss_body(conf_cls_ref, lab_ref, scal_ref, out1_ref, out2_ref, acc_ref):
    b = pl.program_id(0)
    t = pl.program_id(1)
    first = jnp.logical_and(b == 0, t == 0)
    last = jnp.logical_and(b == pl.num_programs(0) - 1,
                           t == pl.num_programs(1) - 1)

    @pl.when(first)
    def _():
        acc_ref[0] = 0.0

    lab = lab_ref[0]
    posb = lab > 0.0
    pos = posb.astype(jnp.float32)
    neg = (lab == 0.0).astype(jnp.float32)

    x = conf_cls_ref[0]
    p = jnp.clip(1.0 / (1.0 + jnp.exp(-x)), EPS, 1.0 - EPS)
    col = jax.lax.broadcasted_iota(jnp.int32, (T2, 81), 1)
    col0 = col == 0
    colf = col.astype(jnp.float32)
    yf = jnp.where(col0, pos, (lab == colf).astype(jnp.float32))
    w = pos + jnp.where(col0, 0.5 * neg, 0.0)
    q = jnp.where(yf > 0.0, p, 1.0 - p)
    acc_ref[0] += jnp.sum(w * (-jnp.log(q)))

    @pl.when(last)
    def _():
        v = scal_ref[...]
        num_pos = jnp.maximum(1.0, v[0:1, 1:2])
        out1_ref[...] = v[0:1, 0:1] / (num_pos * 4.0)
        out2_ref[...] = jnp.full((1, 1), acc_ref[0]) / (2.0 * num_pos)


def kernel(confidence, predicted_locations, gts, counts, anchors):
    B, A, NC1 = confidence.shape

    corners = jnp.concatenate(
        [anchors[:, :2] - anchors[:, 2:] * 0.5,
         anchors[:, :2] + anchors[:, 2:] * 0.5], axis=1)
    anchors8 = jnp.concatenate([corners.T, anchors.T], axis=0)
    ploc_l = jnp.swapaxes(predicted_locations, 1, 2)

    valid = (jnp.arange(NGT)[None, :] < counts[:, None])
    sent = jnp.array([-5.0, -5.0, -4.0, -4.0], jnp.float32)
    gbox = jnp.where(valid[:, :, None], gts[:, :, :4], sent[None, None, :])
    garea = ((gbox[:, :, 2] - gbox[:, :, 0])
             * (gbox[:, :, 3] - gbox[:, :, 1]) + 1e-9)
    glab = jnp.where(valid, gts[:, :, 4], 0.0)
    gt_t = jnp.concatenate(
        [jnp.swapaxes(gbox, 1, 2), garea[:, None, :], glab[:, None, :],
         jnp.zeros((B, 2, NGT), jnp.float32)], axis=1)

    conf_l, scal = pl.pallas_call(
        _match_body,
        grid=(B, A // T1),
        in_specs=[
            pl.BlockSpec((8, T1), lambda b, t: (0, t)),
            pl.BlockSpec((1, 8, NGT), lambda b, t: (b, 0, 0),
                         memory_space=pltpu.SMEM),
            pl.BlockSpec((1, 4, T1), lambda b, t: (b, 0, t)),
        ],
        out_specs=[
            pl.BlockSpec((1, 1, T1), lambda b, t: (b, 0, t)),
            pl.BlockSpec((1, 2), lambda b, t: (0, 0)),
        ],
        out_shape=[
            jax.ShapeDtypeStruct((B, 1, A), jnp.float32),
            jax.ShapeDtypeStruct((1, 2), jnp.float32),
        ],
        scratch_shapes=[pltpu.SMEM((2,), jnp.float32)],
        interpret=_INTERPRET,
    )(anchors8, gt_t, ploc_l)

    lab_s = jnp.swapaxes(conf_l, 1, 2)

    out1, out2 = pl.pallas_call(
        _loss_body,
        grid=(B, A // T2),
        in_specs=[
            pl.BlockSpec((1, T2, NC1), lambda b, t: (b, t, 0)),
            pl.BlockSpec((1, T2, 1), lambda b, t: (b, t, 0)),
            pl.BlockSpec((1, 2), lambda b, t: (0, 0)),
        ],
        out_specs=[
            pl.BlockSpec((1, 1), lambda b, t: (0, 0)),
            pl.BlockSpec((1, 1), lambda b, t: (0, 0)),
        ],
        out_shape=[
            jax.ShapeDtypeStruct((1, 1), jnp.float32),
            jax.ShapeDtypeStruct((1, 1), jnp.float32),
        ],
        scratch_shapes=[pltpu.SMEM((1,), jnp.float32)],
        interpret=_INTERPRET,
    )(confidence, lab_s, scal)

    return out1[0, 0], out2[0, 0]


# speedup vs baseline: 13.8160x; 1.1366x over previous
import jax
import jax.numpy as jnp
from jax.experimental import pallas as pl
from jax.experimental.pallas import tpu as pltpu

POS_TH = 0.5
NEG_TH = 0.4
EPS = 1e-7
LOG_EPS = -16.11809565095832
BETA = 1.0 / 9.0

_INTERPRET = False

T1 = 4096
T2 = 4096
NGT = 50
NC = 81


def _match_body(a_ref, gt_ref, ploc_ref, conf_ref, scal_ref, acc_ref):
    b = pl.program_id(0)
    t = pl.program_id(1)
    first = jnp.logical_and(b == 0, t == 0)
    last = jnp.logical_and(b == pl.num_programs(0) - 1,
                           t == pl.num_programs(1) - 1)

    @pl.when(first)
    def _():
        acc_ref[0] = 0.0
        acc_ref[1] = 0.0

    ax1 = a_ref[0:1, :]
    ay1 = a_ref[1:2, :]
    ax2 = a_ref[2:3, :]
    ay2 = a_ref[3:4, :]
    area_a = (ax2 - ax1) * (ay2 - ay1)

    z = jnp.zeros((1, T1), jnp.float32)
    best = jnp.full((1, T1), -1.0, jnp.float32)
    lab = z
    mx1 = z
    my1 = z
    mx2 = z
    my2 = z
    for g in range(NGT):
        gx1 = gt_ref[0, 0, g]
        gy1 = gt_ref[0, 1, g]
        gx2 = gt_ref[0, 2, g]
        gy2 = gt_ref[0, 3, g]
        garea = gt_ref[0, 4, g]
        glab = gt_ref[0, 5, g]
        wx = jnp.maximum(jnp.minimum(ax2, gx2) - jnp.maximum(ax1, gx1), 0.0)
        wy = jnp.maximum(jnp.minimum(ay2, gy2) - jnp.maximum(ay1, gy1), 0.0)
        inter = wx * wy
        iou = inter / (area_a + (garea - inter))
        upd = iou > best
        best = jnp.where(upd, iou, best)
        lab = jnp.where(upd, glab, lab)
        mx1 = jnp.where(upd, gx1, mx1)
        my1 = jnp.where(upd, gy1, my1)
        mx2 = jnp.where(upd, gx2, mx2)
        my2 = jnp.where(upd, gy2, my2)

    conf = jnp.where(best < POS_TH, -1.0, lab)
    conf = jnp.where(best < NEG_TH, 0.0, conf)
    conf_ref[0, 0:1, :] = conf
    pos = (conf > 0.0).astype(jnp.float32)

    acx = a_ref[4:5, :]
    acy = a_ref[5:6, :]
    aw = a_ref[6:7, :]
    ah = a_ref[7:8, :]
    l0 = ((mx1 + mx2) * 0.5 - acx) / (aw * 0.1)
    l1 = ((my1 + my2) * 0.5 - acy) / (ah * 0.1)
    l2 = jnp.log(jnp.maximum(mx2 - mx1, 1e-6) / aw) * 5.0
    l3 = jnp.log(jnp.maximum(my2 - my1, 1e-6) / ah) * 5.0

    sl1 = z
    for i, l in enumerate((l0, l1, l2, l3)):
        n = jnp.abs(ploc_ref[0, i:i + 1, :] - l)
        sl1 = sl1 + jnp.where(n < BETA, n * n * (0.5 / BETA), n - 0.5 * BETA)
    acc_ref[0] += jnp.sum(sl1 * pos)
    acc_ref[1] += jnp.sum(pos)

    @pl.when(last)
    def _():
        scal_ref[0:1, 0:1] = jnp.full((1, 1), acc_ref[0])
        scal_ref[0:1, 1:2] = jnp.full((1, 1), acc_ref[1])


def _loss_body(conf_cls_ref, lab_ref, scal_ref, out1_ref, out2_ref, acc_ref):
    b = pl.program_id(0)
    t = pl.program_id(1)
    first = jnp.logical_and(b == 0, t == 0)
    last = jnp.logical_and(b == pl.num_programs(0) - 1,
                           t == pl.num_programs(1) - 1)

    @pl.when(first)
    def _():
        acc_ref[0] = 0.0

    lab = lab_ref[0]
    labb = lab + jnp.zeros((T2, NC), jnp.float32)

    x = conf_cls_ref[0]
    p = 1.0 / (1.0 + jnp.exp(-x))
    col = jax.lax.broadcasted_iota(jnp.int32, (T2, NC), 1)
    col0 = col == 0
    colf = col.astype(jnp.float32)
    posf = jnp.clip(labb, 0.0, 1.0)
    eqf = (labb == colf).astype(jnp.float32)
    yf = jnp.where(col0, posf, eqf)
    negf = jnp.clip(1.0 - jnp.abs(labb), 0.0, 1.0)
    w = posf + jnp.where(col0, 0.5 * negf, 0.0)
    q = jnp.where(yf > 0.0, p, 1.0 - p)
    acc_ref[0] += jnp.sum(w * jnp.maximum(jnp.log(q), LOG_EPS))

    @pl.when(last)
    def _():
        v = scal_ref[...]
        num_pos = jnp.maximum(1.0, v[0:1, 1:2])
        out1_ref[...] = v[0:1, 0:1] / (num_pos * 4.0)
        out2_ref[...] = jnp.full((1, 1), -acc_ref[0]) / (2.0 * num_pos)


def kernel(confidence, predicted_locations, gts, counts, anchors):
    B, A, NC1 = confidence.shape

    corners = jnp.concatenate(
        [anchors[:, :2] - anchors[:, 2:] * 0.5,
         anchors[:, :2] + anchors[:, 2:] * 0.5], axis=1)
    anchors8 = jnp.concatenate([corners.T, anchors.T], axis=0)
    ploc_l = jnp.swapaxes(predicted_locations, 1, 2)

    valid = (jnp.arange(NGT)[None, :] < counts[:, None])
    sent = jnp.array([-5.0, -5.0, -4.0, -4.0], jnp.float32)
    gbox = jnp.where(valid[:, :, None], gts[:, :, :4], sent[None, None, :])
    garea = ((gbox[:, :, 2] - gbox[:, :, 0])
             * (gbox[:, :, 3] - gbox[:, :, 1]) + 1e-9)
    glab = jnp.where(valid, gts[:, :, 4], 0.0)
    gt_t = jnp.concatenate(
        [jnp.swapaxes(gbox, 1, 2), garea[:, None, :], glab[:, None, :],
         jnp.zeros((B, 2, NGT), jnp.float32)], axis=1)

    conf_l, scal = pl.pallas_call(
        _match_body,
        grid=(B, A // T1),
        in_specs=[
            pl.BlockSpec((8, T1), lambda b, t: (0, t)),
            pl.BlockSpec((1, 8, NGT), lambda b, t: (b, 0, 0),
                         memory_space=pltpu.SMEM),
            pl.BlockSpec((1, 4, T1), lambda b, t: (b, 0, t)),
        ],
        out_specs=[
            pl.BlockSpec((1, 1, T1), lambda b, t: (b, 0, t)),
            pl.BlockSpec((1, 2), lambda b, t: (0, 0)),
        ],
        out_shape=[
            jax.ShapeDtypeStruct((B, 1, A), jnp.float32),
            jax.ShapeDtypeStruct((1, 2), jnp.float32),
        ],
        scratch_shapes=[pltpu.SMEM((2,), jnp.float32)],
        interpret=_INTERPRET,
    )(anchors8, gt_t, ploc_l)

    lab_s = jnp.swapaxes(conf_l, 1, 2)

    out1, out2 = pl.pallas_call(
        _loss_body,
        grid=(B, A // T2),
        in_specs=[
            pl.BlockSpec((1, T2, NC1), lambda b, t: (b, t, 0)),
            pl.BlockSpec((1, T2, 1), lambda b, t: (b, t, 0)),
            pl.BlockSpec((1, 2), lambda b, t: (0, 0)),
        ],
        out_specs=[
            pl.BlockSpec((1, 1), lambda b, t: (0, 0)),
            pl.BlockSpec((1, 1), lambda b, t: (0, 0)),
        ],
        out_shape=[
            jax.ShapeDtypeStruct((1, 1), jnp.float32),
            jax.ShapeDtypeStruct((1, 1), jnp.float32),
        ],
        scratch_shapes=[pltpu.SMEM((1,), jnp.float32)],
        interpret=_INTERPRET,
    )(confidence, lab_s, scal)

    return out1[0, 0], out2[0, 0]
